# Initial kernel scaffold; baseline (speedup 1.0000x reference)
import jax
import jax.numpy as jnp
from jax.experimental import pallas as pl


def kernel(attn, relative_position_bias_table, relative_position_index, n):
    raise NotImplementedError("write your pallas kernel here")


# speedup vs baseline: 2.0752x; 2.0752x over previous
import functools

import jax
import jax.numpy as jnp
from jax import lax
from jax.experimental import pallas as pl
from jax.experimental.pallas import tpu as pltpu
from jax.experimental.pallas import tpu_sc as plsc

B, H, N = 32, 16, 512
NN = N * N
TABLE = 3375

NC, NS = 2, 16
NW = NC * NS
PER_W = NN // NW


def _sc_gather(table_flat, idx16):
    mesh = plsc.VectorSubcoreMesh(core_axis_name="c", subcore_axis_name="s")

    @functools.partial(
        pl.kernel,
        out_type=jax.ShapeDtypeStruct((H, NN), jnp.float32),
        mesh=mesh,
        scratch_types=[
            pltpu.VMEM((TABLE * H,), jnp.float32),
            pltpu.VMEM((PER_W,), jnp.int32),
            pltpu.VMEM((PER_W,), jnp.float32),
        ],
        compiler_params=pltpu.CompilerParams(needs_layout_passes=False),
    )
    def k(table_hbm, idx_hbm, out_hbm, table_v, idx_v, row_v):
        wid = lax.axis_index("s") * NC + lax.axis_index("c")
        base = wid * PER_W
        pltpu.sync_copy(table_hbm, table_v)
        pltpu.sync_copy(idx_hbm.at[pl.ds(base, PER_W)], idx_v)
        for h in range(H):
            def body(v4, carry, h=h):
                for u in range(4):
                    off = v4 * 64 + u * 16
                    ids = idx_v[pl.ds(off, 16)] + h
                    row_v[pl.ds(off, 16)] = plsc.load_gather(table_v, [ids])
                return carry
            lax.fori_loop(0, PER_W // 64, body, 0)
            pltpu.sync_copy(row_v, out_hbm.at[h, pl.ds(base, PER_W)])

    return k(table_flat, idx16)


def _add_body(a_ref, b_ref, o_ref):
    o_ref[0, 0] = a_ref[0, 0] + b_ref[0]


def _broadcast_add(attn, bias3):
    return pl.pallas_call(
        _add_body,
        grid=(H, B),
        in_specs=[
            pl.BlockSpec((1, 1, N, N), lambda h, b: (b, h, 0, 0)),
            pl.BlockSpec((1, N, N), lambda h, b: (h, 0, 0)),
        ],
        out_specs=pl.BlockSpec((1, 1, N, N), lambda h, b: (b, h, 0, 0)),
        out_shape=jax.ShapeDtypeStruct((B, H, N, N), jnp.float32),
    )(attn, bias3)


def kernel(attn, relative_position_bias_table, relative_position_index, n):
    idx16 = (relative_position_index.reshape(-1) + (n - N)) * H
    idx16 = idx16.astype(jnp.int32)
    bias = _sc_gather(relative_position_bias_table.reshape(-1), idx16)
    return _broadcast_add(attn, bias.reshape(H, N, N))


# speedup vs baseline: 2.8941x; 1.3946x over previous
import functools

import jax
import jax.numpy as jnp
from jax import lax
from jax.experimental import pallas as pl
from jax.experimental.pallas import tpu as pltpu
from jax.experimental.pallas import tpu_sc as plsc

B, H, N = 32, 16, 512
NN = N * N
TABLE = 3375

NC, NS = 2, 16
NW = NC * NS
PER_W = NN // NW


def _sc_gather(table_flat, idx16):
    mesh = plsc.VectorSubcoreMesh(core_axis_name="c", subcore_axis_name="s")

    @functools.partial(
        pl.kernel,
        out_type=jax.ShapeDtypeStruct((H, NN), jnp.float32),
        mesh=mesh,
        scratch_types=[
            pltpu.VMEM((TABLE * H,), jnp.float32),
            pltpu.VMEM((PER_W,), jnp.int32),
            pltpu.VMEM((PER_W,), jnp.float32),
        ],
        compiler_params=pltpu.CompilerParams(needs_layout_passes=False),
    )
    def k(table_hbm, idx_hbm, out_hbm, table_v, idx_v, row_v):
        wid = lax.axis_index("s") * NC + lax.axis_index("c")
        base = wid * PER_W
        pltpu.sync_copy(table_hbm, table_v)
        pltpu.sync_copy(idx_hbm.at[pl.ds(base, PER_W)], idx_v)
        for h in range(H):
            def body(v4, carry, h=h):
                for u in range(4):
                    off = v4 * 64 + u * 16
                    ids = idx_v[pl.ds(off, 16)] + h
                    row_v[pl.ds(off, 16)] = plsc.load_gather(table_v, [ids])
                return carry
            lax.fori_loop(0, PER_W // 64, body, 0)
            pltpu.sync_copy(row_v, out_hbm.at[h, pl.ds(base, PER_W)])

    return k(table_flat, idx16)


HB = 4


def _add_body(a_ref, b_ref, o_ref):
    o_ref[0] = a_ref[0] + b_ref[...]


def _broadcast_add(attn, bias3):
    return pl.pallas_call(
        _add_body,
        grid=(H // HB, B),
        in_specs=[
            pl.BlockSpec((1, HB, N, N), lambda h, b: (b, h, 0, 0)),
            pl.BlockSpec((HB, N, N), lambda h, b: (h, 0, 0)),
        ],
        out_specs=pl.BlockSpec((1, HB, N, N), lambda h, b: (b, h, 0, 0)),
        out_shape=jax.ShapeDtypeStruct((B, H, N, N), jnp.float32),
    )(attn, bias3)


def kernel(attn, relative_position_bias_table, relative_position_index, n):
    idx16 = (relative_position_index.reshape(-1) + (n - N)) * H
    idx16 = idx16.astype(jnp.int32)
    bias = _sc_gather(relative_position_bias_table.reshape(-1), idx16)
    return _broadcast_add(attn, bias.reshape(H, N, N))


# speedup vs baseline: 2.9271x; 1.0114x over previous
import functools

import jax
import jax.numpy as jnp
from jax import lax
from jax.experimental import pallas as pl
from jax.experimental.pallas import tpu as pltpu
from jax.experimental.pallas import tpu_sc as plsc

B, H, N = 32, 16, 512
NN = N * N
TABLE = 3375

NC, NS = 2, 16
NW = NC * NS
PER_W = NN // NW


def _sc_gather(table_flat, idx16):
    mesh = plsc.VectorSubcoreMesh(core_axis_name="c", subcore_axis_name="s")

    @functools.partial(
        pl.kernel,
        out_type=jax.ShapeDtypeStruct((H, NN), jnp.float32),
        mesh=mesh,
        scratch_types=[
            pltpu.VMEM((TABLE * H,), jnp.float32),
            pltpu.VMEM((PER_W,), jnp.int32),
            pltpu.VMEM((PER_W,), jnp.float32),
        ],
        compiler_params=pltpu.CompilerParams(needs_layout_passes=False),
    )
    def k(table_hbm, idx_hbm, out_hbm, table_v, idx_v, row_v):
        wid = lax.axis_index("s") * NC + lax.axis_index("c")
        base = wid * PER_W
        pltpu.sync_copy(table_hbm, table_v)
        pltpu.sync_copy(idx_hbm.at[pl.ds(base, PER_W)], idx_v)
        for h in range(H):
            def body(v4, carry, h=h):
                for u in range(4):
                    off = v4 * 64 + u * 16
                    ids = idx_v[pl.ds(off, 16)] + h
                    row_v[pl.ds(off, 16)] = plsc.load_gather(table_v, [ids])
                return carry
            lax.fori_loop(0, PER_W // 64, body, 0)
            pltpu.sync_copy(row_v, out_hbm.at[h, pl.ds(base, PER_W)])

    return k(table_flat, idx16)


HB = 8


def _add_body(a_ref, b_ref, o_ref):
    o_ref[0] = a_ref[0] + b_ref[...]


def _broadcast_add(attn, bias3):
    return pl.pallas_call(
        _add_body,
        grid=(H // HB, B),
        in_specs=[
            pl.BlockSpec((1, HB, N, N), lambda h, b: (b, h, 0, 0)),
            pl.BlockSpec((HB, N, N), lambda h, b: (h, 0, 0)),
        ],
        out_specs=pl.BlockSpec((1, HB, N, N), lambda h, b: (b, h, 0, 0)),
        out_shape=jax.ShapeDtypeStruct((B, H, N, N), jnp.float32),
    )(attn, bias3)


def kernel(attn, relative_position_bias_table, relative_position_index, n):
    idx16 = (relative_position_index.reshape(-1) + (n - N)) * H
    idx16 = idx16.astype(jnp.int32)
    bias = _sc_gather(relative_position_bias_table.reshape(-1), idx16)
    return _broadcast_add(attn, bias.reshape(H, N, N))


# speedup vs baseline: 3.2880x; 1.1233x over previous
import functools

import jax
import jax.numpy as jnp
from jax import lax
from jax.experimental import pallas as pl
from jax.experimental.pallas import tpu as pltpu
from jax.experimental.pallas import tpu_sc as plsc

B, H, N = 32, 16, 512
NN = N * N
TABLE = 3375

NC, NS = 2, 16
NW = NC * NS
PER_W = NN // NW


def _sc_gather(table_flat, idx16):
    mesh = plsc.VectorSubcoreMesh(core_axis_name="c", subcore_axis_name="s")

    CH = 2048
    NCH = PER_W // CH

    @functools.partial(
        pl.kernel,
        out_type=jax.ShapeDtypeStruct((H, NN), jnp.float32),
        mesh=mesh,
        scratch_types=[
            pltpu.VMEM((TABLE * H,), jnp.float32),
            pltpu.VMEM((PER_W,), jnp.int32),
            pltpu.VMEM((H * CH,), jnp.float32),
            pltpu.VMEM((H * CH,), jnp.float32),
            pltpu.SemaphoreType.DMA,
        ],
        compiler_params=pltpu.CompilerParams(needs_layout_passes=False),
    )
    def k(table_hbm, idx_hbm, out_hbm, table_v, idx_v, buf_a, buf_b, sem):
        wid = lax.axis_index("s") * NC + lax.axis_index("c")
        base = wid * PER_W
        pltpu.sync_copy(table_hbm, table_v)
        pltpu.sync_copy(idx_hbm.at[pl.ds(base, PER_W)], idx_v)
        bufs = [buf_a, buf_b]
        pending = [[], []]
        for c in range(NCH):
            buf = bufs[c % 2]
            for cp in pending[c % 2]:
                cp.wait()
            pending[c % 2] = []

            @plsc.parallel_loop(0, CH // 16, unroll=2)
            def body(v, c=c, buf=buf):
                off = v * 16
                ids = idx_v[pl.ds(c * CH + off, 16)]
                for h in range(H):
                    buf[pl.ds(h * CH + off, 16)] = plsc.load_gather(
                        table_v, [ids + h])

            for h in range(H):
                pending[c % 2].append(pltpu.async_copy(
                    buf.at[pl.ds(h * CH, CH)],
                    out_hbm.at[h, pl.ds(base + c * CH, CH)],
                    sem))
        for cps in pending:
            for cp in cps:
                cp.wait()

    return k(table_flat, idx16)


HB = 8


def _add_body(a_ref, b_ref, o_ref):
    o_ref[0] = a_ref[0] + b_ref[...]


def _broadcast_add(attn, bias3):
    return pl.pallas_call(
        _add_body,
        grid=(H // HB, B),
        in_specs=[
            pl.BlockSpec((1, HB, N, N), lambda h, b: (b, h, 0, 0)),
            pl.BlockSpec((HB, N, N), lambda h, b: (h, 0, 0)),
        ],
        out_specs=pl.BlockSpec((1, HB, N, N), lambda h, b: (b, h, 0, 0)),
        out_shape=jax.ShapeDtypeStruct((B, H, N, N), jnp.float32),
    )(attn, bias3)


def kernel(attn, relative_position_bias_table, relative_position_index, n):
    idx16 = (relative_position_index.reshape(-1) + (n - N)) * H
    idx16 = idx16.astype(jnp.int32)
    bias = _sc_gather(relative_position_bias_table.reshape(-1), idx16)
    return _broadcast_add(attn, bias.reshape(H, N, N))


# speedup vs baseline: 3.4254x; 1.0418x over previous
import functools

import jax
import jax.numpy as jnp
from jax import lax
from jax.experimental import pallas as pl
from jax.experimental.pallas import tpu as pltpu
from jax.experimental.pallas import tpu_sc as plsc

B, H, N = 32, 16, 512
NN = N * N
TABLE = 3375

NC, NS = 2, 16
NW = NC * NS
PER_W = NN // NW


def _sc_gather(table_flat, idx16):
    mesh = plsc.VectorSubcoreMesh(core_axis_name="c", subcore_axis_name="s")

    CH = 2048
    NCH = PER_W // CH

    CHG = CH // 128

    @functools.partial(
        pl.kernel,
        out_type=jax.ShapeDtypeStruct((H, NN // 128, 128), jnp.float32),
        mesh=mesh,
        scratch_types=[
            pltpu.VMEM((TABLE * H,), jnp.float32),
            pltpu.VMEM((PER_W,), jnp.int32),
            pltpu.VMEM((H, CHG, 128), jnp.float32),
            pltpu.VMEM((H, CHG, 128), jnp.float32),
            pltpu.SemaphoreType.DMA,
        ],
        compiler_params=pltpu.CompilerParams(needs_layout_passes=False),
    )
    def k(table_hbm, idx_hbm, out_hbm, table_v, idx_v, buf_a, buf_b, sem):
        wid = lax.axis_index("s") * NC + lax.axis_index("c")
        base = wid * PER_W
        pltpu.sync_copy(table_hbm, table_v)
        pltpu.sync_copy(idx_hbm.at[pl.ds(base, PER_W)], idx_v)
        bufs = [buf_a, buf_b]
        pending = [[], []]
        for c in range(NCH):
            buf = bufs[c % 2]
            for cp in pending[c % 2]:
                cp.wait()
            pending[c % 2] = []

            @plsc.parallel_loop(0, CH // 16, unroll=2)
            def body(v, c=c, buf=buf):
                g = v // 8
                l0 = (v % 8) * 16
                ids = idx_v[pl.ds(c * CH + v * 16, 16)]
                for h in range(H):
                    buf[h, g, pl.ds(l0, 16)] = plsc.load_gather(
                        table_v, [ids + h])

            gbase = wid * (PER_W // 128) + c * CHG
            for h in range(H):
                pending[c % 2].append(pltpu.async_copy(
                    buf.at[h],
                    out_hbm.at[h, pl.ds(gbase, CHG)],
                    sem))
        for cps in pending:
            for cp in cps:
                cp.wait()

    return k(table_flat, idx16)


HB = 8


def _add_body(a_ref, b_ref, o_ref, scr_ref):
    @pl.when(pl.program_id(1) == 0)
    def _():
        scr_ref[...] = b_ref[...].reshape(HB, N, N)

    o_ref[0] = a_ref[0] + scr_ref[...]


def _broadcast_add(attn, bias3):
    return pl.pallas_call(
        _add_body,
        grid=(H // HB, B),
        in_specs=[
            pl.BlockSpec((1, HB, N, N), lambda h, b: (b, h, 0, 0)),
            pl.BlockSpec((HB, NN // 128, 128), lambda h, b: (h, 0, 0)),
        ],
        out_specs=pl.BlockSpec((1, HB, N, N), lambda h, b: (b, h, 0, 0)),
        out_shape=jax.ShapeDtypeStruct((B, H, N, N), jnp.float32),
        scratch_shapes=[pltpu.VMEM((HB, N, N), jnp.float32)],
    )(attn, bias3)


def kernel(attn, relative_position_bias_table, relative_position_index, n):
    idx16 = (relative_position_index.reshape(-1) + (n - N)) * H
    idx16 = idx16.astype(jnp.int32)
    bias = _sc_gather(relative_position_bias_table.reshape(-1), idx16)
    return _broadcast_add(attn, bias)
